# speedup vs baseline: 1.4630x; 1.4630x over previous
import functools

import jax
import jax.numpy as jnp
from jax import lax
from jax.experimental import pallas as pl
from jax.experimental.pallas import tpu as pltpu
from jax.experimental.pallas import tpu_sc as plsc

_B = 16384
_D = 128
_NC = 2
_NS = 16
_NW = _NC * _NS
_BPW = _B // _NW
_CHUNK = 128
_NCHUNK = _BPW // _CHUNK

_mesh = plsc.VectorSubcoreMesh(core_axis_name="c", subcore_axis_name="s")


@functools.partial(
    pl.kernel,
    mesh=_mesh,
    out_type=(
        jax.ShapeDtypeStruct((_B, _D), jnp.float32),
        jax.ShapeDtypeStruct((_B, _D), jnp.float32),
    ),
    scratch_types=[
        pltpu.VMEM((_NCHUNK, _CHUNK), jnp.int32),
        pltpu.VMEM((_CHUNK, _D), jnp.float32),
        pltpu.VMEM((_CHUNK, _D), jnp.float32),
        pltpu.SemaphoreType.DMA,
        pltpu.SemaphoreType.DMA,
    ],
)
def _gather2(idx_hbm, feat_hbm, embed_hbm, out_feat, out_embed,
             idx_v, fbuf, ebuf, fsem, esem):
    wid = lax.axis_index("s") * _NC + lax.axis_index("c")
    base = wid * _BPW
    pltpu.sync_copy(idx_hbm.at[wid], idx_v)
    for ci in range(_NCHUNK):
        cidx = idx_v.at[ci]
        fcp = pltpu.async_copy(feat_hbm.at[cidx], fbuf, fsem)
        ecp = pltpu.async_copy(embed_hbm.at[cidx], ebuf, esem)
        off = base + ci * _CHUNK
        fcp.wait()
        pltpu.sync_copy(fbuf, out_feat.at[pl.ds(off, _CHUNK)])
        ecp.wait()
        pltpu.sync_copy(ebuf, out_embed.at[pl.ds(off, _CHUNK)])


@jax.jit
def kernel(indices, query_feat, query_embed):
    idx = indices.astype(jnp.int32).reshape(_NW, _NCHUNK, _CHUNK)
    return _gather2(idx, query_feat, query_embed)


# speedup vs baseline: 1.6161x; 1.1046x over previous
import functools

import jax
import jax.numpy as jnp
from jax import lax
from jax.experimental import pallas as pl
from jax.experimental.pallas import tpu as pltpu
from jax.experimental.pallas import tpu_sc as plsc

_B = 16384
_D = 128
_NC = 2
_NS = 16
_NW = _NC * _NS
_BPW = _B // _NW
_CHUNK = 128
_NCHUNK = _BPW // _CHUNK
_NBUF = 3

_mesh = plsc.VectorSubcoreMesh(core_axis_name="c", subcore_axis_name="s")

_scratch = [pltpu.VMEM((_NCHUNK, _CHUNK), jnp.int32)]
_scratch += [pltpu.VMEM((_CHUNK, _D), jnp.float32) for _ in range(2 * _NBUF)]
_scratch += [pltpu.SemaphoreType.DMA for _ in range(2 * _NBUF)]


@functools.partial(
    pl.kernel,
    mesh=_mesh,
    out_type=(
        jax.ShapeDtypeStruct((_B, _D), jnp.float32),
        jax.ShapeDtypeStruct((_B, _D), jnp.float32),
    ),
    scratch_types=_scratch,
)
def _gather2(idx_hbm, feat_hbm, embed_hbm, out_feat, out_embed,
             idx_v, *bufs_and_sems):
    fbufs = bufs_and_sems[0:_NBUF]
    ebufs = bufs_and_sems[_NBUF:2 * _NBUF]
    fsems = bufs_and_sems[2 * _NBUF:3 * _NBUF]
    esems = bufs_and_sems[3 * _NBUF:4 * _NBUF]

    wid = lax.axis_index("s") * _NC + lax.axis_index("c")
    base = wid * _BPW
    pltpu.sync_copy(idx_hbm.at[wid], idx_v)

    gath_f, gath_e, st_f, st_e = {}, {}, {}, {}
    for ci in range(min(_NBUF, _NCHUNK)):
        b = ci % _NBUF
        gath_f[ci] = pltpu.async_copy(feat_hbm.at[idx_v.at[ci]], fbufs[b], fsems[b])
        gath_e[ci] = pltpu.async_copy(embed_hbm.at[idx_v.at[ci]], ebufs[b], esems[b])

    for ci in range(_NCHUNK):
        b = ci % _NBUF
        off = base + ci * _CHUNK
        gath_f[ci].wait()
        st_f[ci] = pltpu.async_copy(fbufs[b], out_feat.at[pl.ds(off, _CHUNK)], fsems[b])
        gath_e[ci].wait()
        st_e[ci] = pltpu.async_copy(ebufs[b], out_embed.at[pl.ds(off, _CHUNK)], esems[b])
        nci = ci + _NBUF
        if nci < _NCHUNK:
            st_f[ci].wait()
            gath_f[nci] = pltpu.async_copy(feat_hbm.at[idx_v.at[nci]], fbufs[b], fsems[b])
            st_e[ci].wait()
            gath_e[nci] = pltpu.async_copy(embed_hbm.at[idx_v.at[nci]], ebufs[b], esems[b])

    for ci in range(max(0, _NCHUNK - _NBUF), _NCHUNK):
        st_f[ci].wait()
        st_e[ci].wait()


@jax.jit
def kernel(indices, query_feat, query_embed):
    idx = indices.astype(jnp.int32).reshape(_NW, _NCHUNK, _CHUNK)
    return _gather2(idx, query_feat, query_embed)
